# baseline (speedup 1.0000x reference)
import functools

import numpy as np
import jax
import jax.numpy as jnp
from jax import lax
from jax.experimental import pallas as pl
from jax.experimental.pallas import tpu as pltpu
from jax.experimental.pallas import tpu_sc as plsc

_VOCAB = 1000000
_EMB = 64
_SEQ = 2048
_BATCH = 4
_NTOK = _BATCH * _SEQ

_NC = 2
_NS = 16
_NW = _NC * _NS
_LANES = 16

_WCOLS = 640
_NWIN = -(-_VOCAB // _WCOLS)
_WPT = (-(-_NWIN // _NW) + 1) // 2 * 2
_LAST_COL0 = -(-(_VOCAB - _WCOLS) // 128) * 128
_OUT_ROWS = _NTOK


def _pe_host(seq: int, d: int) -> np.ndarray:
    pos = np.arange(seq, dtype=np.float64)[:, None]
    index = np.arange(d, dtype=np.float64)[None, :]
    tmp = pos / np.power(10000.0, index / float(d))
    pe = np.zeros((seq, d), dtype=np.float64)
    pe[:, 0::2] = np.sin(tmp[:, 0::2])
    pe[:, 1::2] = np.cos(tmp[:, 1::2])
    return pe.astype(np.float32)


_PE = _pe_host(_SEQ, _EMB)

_mesh = plsc.VectorSubcoreMesh(core_axis_name="c", subcore_axis_name="s")


@functools.partial(
    pl.kernel,
    mesh=_mesh,
    compiler_params=pltpu.CompilerParams(needs_layout_passes=False),
    out_type=jax.ShapeDtypeStruct((_OUT_ROWS, 128), jnp.float32),
    scratch_types=[
        pltpu.VMEM((_NTOK,), jnp.int32),
        pltpu.VMEM((_NTOK,), jnp.int32),
        pltpu.VMEM((_NTOK,), jnp.int32),
        pltpu.VMEM((_NTOK,), jnp.int32),
        pltpu.VMEM((_EMB, _WCOLS), jnp.float32),
        pltpu.VMEM((_EMB, _WCOLS), jnp.float32),
        pltpu.VMEM((2 * _LANES, 128), jnp.float32),
        pltpu.VMEM((2, _LANES), jnp.int32),
        pltpu.SemaphoreType.DMA,
        pltpu.SemaphoreType.DMA,
        pltpu.SemaphoreType.DMA,
    ],
)
def _gather_sc(tablet_hbm, idx_hbm, out_hbm, idx_v, vlist, tlist, wpack,
               win0, win1, staging, srow, sem0, sem1, sem2):
    wid = lax.axis_index("s") * _NC + lax.axis_index("c")

    def fire(k, win, sem):
        w_glob = wid + _NW * k
        col0 = jnp.minimum(w_glob * _WCOLS, _LAST_COL0)
        col0 = pl.multiple_of(col0, 128)
        pltpu.async_copy(tablet_hbm.at[:, pl.ds(col0, _WCOLS)], win, sem)

    fire(0, win0, sem0)
    fire(1, win1, sem1)

    pltpu.sync_copy(idx_hbm, idx_v)

    lane = lax.broadcasted_iota(jnp.int32, (_LANES,), 0)

    def bucket_body(c, cnt):
        v = idx_v[pl.ds(c * _LANES, _LANES)]
        m = ((v // _WCOLS) & (_NW - 1)) == wid
        plsc.store_compressed(vlist.at[pl.ds(cnt, _LANES)], v, mask=m)
        plsc.store_compressed(
            tlist.at[pl.ds(cnt, _LANES)], c * _LANES + lane, mask=m
        )
        npop = plsc.all_reduce_population_count(m)
        return cnt + npop[0]

    cnt = lax.fori_loop(0, _NTOK // _LANES, bucket_body, 0)
    nchunks = (cnt + _LANES - 1) // _LANES

    def process(k, win, sem, gb):
        pltpu.make_async_copy(
            tablet_hbm.at[:, pl.ds(0, _WCOLS)], win, sem
        ).wait()
        w_glob = wid + _NW * k
        dma_col0 = jnp.minimum(w_glob * _WCOLS, _LAST_COL0)

        def sub_body(c, nw):
            valid = (c * _LANES + lane) < cnt
            v = vlist[pl.ds(c * _LANES, _LANES)]
            t = tlist[pl.ds(c * _LANES, _LANES)]
            m = valid & ((v // _WCOLS) == w_glob)
            packed = (v - dma_col0) | (t << 10)
            plsc.store_compressed(wpack.at[pl.ds(nw, _LANES)], packed, mask=m)
            npop = plsc.all_reduce_population_count(m)
            return nw + npop[0]

        nw = lax.fori_loop(0, nchunks, sub_body, 0)

        def batch_body(b, g):
            slot = g % 2

            @pl.when(g >= 2)
            def _():
                pltpu.make_async_copy(
                    out_hbm.at[pl.ds(0, _LANES)],
                    staging.at[pl.ds(0, _LANES)],
                    sem2,
                ).wait()

            msk = (b * _LANES + lane) < nw
            p = wpack[pl.ds(b * _LANES, _LANES)]
            jv = p & 1023
            tv = p >> 10
            jv = jnp.where(msk, jv, jv[0])
            tv = jnp.where(msk, tv, tv[0])
            srot = slot * _LANES + lane
            for e in range(_EMB):
                g_vals = plsc.load_gather(
                    win, [jnp.full((_LANES,), e, jnp.int32), jv]
                )
                plsc.store_scatter(
                    staging,
                    [srot, jnp.full((_LANES,), e, jnp.int32)],
                    g_vals,
                )
            srow[slot, :] = tv
            pltpu.async_copy(
                staging.at[pl.ds(slot * _LANES, _LANES)],
                out_hbm.at[srow.at[slot]],
                sem2,
            )
            return g + 1

        return lax.fori_loop(0, (nw + _LANES - 1) // _LANES, batch_body, gb)

    def step(i, gb):
        gb = process(2 * i, win0, sem0, gb)

        @pl.when(2 * i + 2 < _WPT)
        def _():
            fire(2 * i + 2, win0, sem0)

        gb = process(2 * i + 1, win1, sem1, gb)

        @pl.when(2 * i + 3 < _WPT)
        def _():
            fire(2 * i + 3, win1, sem1)

        return gb

    gb = lax.fori_loop(0, _WPT // 2, step, 0)

    def drain_body(i, carry):
        pltpu.make_async_copy(
            out_hbm.at[pl.ds(0, _LANES)],
            staging.at[pl.ds(0, _LANES)],
            sem2,
        ).wait()
        return carry

    lax.fori_loop(0, jnp.minimum(gb, 2), drain_body, 0)


def _epilogue_body(raw_ref, idx_ref, pe_ref, out_ref):
    rows = raw_ref[:, :_EMB]
    m = (idx_ref[0, 0, :] != 0).astype(jnp.float32).reshape(-1, 1)
    out_ref[...] = rows * m + pe_ref[...]


_EPI_BLK = 256


def _epilogue(raw, idx3, pe):
    return pl.pallas_call(
        _epilogue_body,
        grid=(_NTOK // _EPI_BLK,),
        in_specs=[
            pl.BlockSpec((_EPI_BLK, 128), lambda b: (b, 0)),
            pl.BlockSpec((1, 1, _EPI_BLK), lambda b: (b, 0, 0)),
            pl.BlockSpec((_EPI_BLK, _EMB), lambda b: (b % (_SEQ // _EPI_BLK), 0)),
        ],
        out_specs=pl.BlockSpec((_EPI_BLK, _EMB), lambda b: (b, 0)),
        out_shape=jax.ShapeDtypeStruct((_NTOK, _EMB), jnp.float32),
    )(raw, idx3, pe)


def kernel(input, table):
    idx_flat = input.reshape(_NTOK)
    raw = _gather_sc(table.T, idx_flat)
    idx3 = idx_flat.reshape(_NTOK // _EPI_BLK, 1, _EPI_BLK)
    pe = jnp.asarray(_PE)
    out = _epilogue(raw, idx3, pe)
    return out.reshape(_BATCH, _SEQ, _EMB)


# speedup vs baseline: 1.0961x; 1.0961x over previous
import functools

import numpy as np
import jax
import jax.numpy as jnp
from jax import lax
from jax.experimental import pallas as pl
from jax.experimental.pallas import tpu as pltpu
from jax.experimental.pallas import tpu_sc as plsc

_VOCAB = 1000000
_EMB = 64
_SEQ = 2048
_BATCH = 4
_NTOK = _BATCH * _SEQ

_NC = 2
_NS = 16
_NW = _NC * _NS
_LANES = 16

_WCOLS = 512
_NWIN = -(-_VOCAB // _WCOLS)
_WPT = (-(-_NWIN // _NW) + 1) // 2 * 2
_LAST_COL0 = -(-(_VOCAB - _WCOLS) // 128) * 128
_OUT_ROWS = _NTOK


def _pe_host(seq: int, d: int) -> np.ndarray:
    pos = np.arange(seq, dtype=np.float64)[:, None]
    index = np.arange(d, dtype=np.float64)[None, :]
    tmp = pos / np.power(10000.0, index / float(d))
    pe = np.zeros((seq, d), dtype=np.float64)
    pe[:, 0::2] = np.sin(tmp[:, 0::2])
    pe[:, 1::2] = np.cos(tmp[:, 1::2])
    return pe.astype(np.float32)


_PE = _pe_host(_SEQ, _EMB)

_mesh = plsc.VectorSubcoreMesh(core_axis_name="c", subcore_axis_name="s")


@functools.partial(
    pl.kernel,
    mesh=_mesh,
    compiler_params=pltpu.CompilerParams(needs_layout_passes=False),
    out_type=jax.ShapeDtypeStruct((_OUT_ROWS, 128), jnp.float32),
    scratch_types=[
        pltpu.VMEM((_NTOK,), jnp.int32),
        pltpu.VMEM((_NTOK,), jnp.int32),
        pltpu.VMEM((_NTOK,), jnp.int32),
        pltpu.VMEM((_NTOK,), jnp.int32),
        pltpu.VMEM((_EMB, _WCOLS), jnp.float32),
        pltpu.VMEM((_EMB, _WCOLS), jnp.float32),
        pltpu.VMEM((2 * _LANES, 128), jnp.float32),
        pltpu.VMEM((2, _LANES), jnp.int32),
        pltpu.SemaphoreType.DMA,
        pltpu.SemaphoreType.DMA,
        pltpu.SemaphoreType.DMA,
    ],
)
def _gather_sc(tablet_hbm, idx_hbm, out_hbm, idx_v, vlist, tlist, wpack,
               win0, win1, staging, srow, sem0, sem1, sem2):
    wid = lax.axis_index("s") * _NC + lax.axis_index("c")

    def fire(k, win, sem):
        w_glob = wid + _NW * k
        col0 = jnp.minimum(w_glob * _WCOLS, _LAST_COL0)
        col0 = pl.multiple_of(col0, 128)
        pltpu.async_copy(tablet_hbm.at[:, pl.ds(col0, _WCOLS)], win, sem)

    fire(0, win0, sem0)
    fire(1, win1, sem1)

    pltpu.sync_copy(idx_hbm, idx_v)

    lane = lax.broadcasted_iota(jnp.int32, (_LANES,), 0)

    def bucket_body(c, cnt):
        v = idx_v[pl.ds(c * _LANES, _LANES)]
        m = ((v >> 9) & (_NW - 1)) == wid
        plsc.store_compressed(vlist.at[pl.ds(cnt, _LANES)], v, mask=m)
        plsc.store_compressed(
            tlist.at[pl.ds(cnt, _LANES)], c * _LANES + lane, mask=m
        )
        npop = plsc.all_reduce_population_count(m)
        return cnt + npop[0]

    cnt = lax.fori_loop(0, _NTOK // _LANES, bucket_body, 0)
    nchunks = (cnt + _LANES - 1) // _LANES

    def process(k, win, sem, gb):
        pltpu.make_async_copy(
            tablet_hbm.at[:, pl.ds(0, _WCOLS)], win, sem
        ).wait()
        w_glob = wid + _NW * k
        dma_col0 = jnp.minimum(w_glob * _WCOLS, _LAST_COL0)

        def sub_body(c, nw):
            valid = (c * _LANES + lane) < cnt
            v = vlist[pl.ds(c * _LANES, _LANES)]
            t = tlist[pl.ds(c * _LANES, _LANES)]
            m = valid & ((v >> 9) == w_glob)
            packed = (v - dma_col0) | (t << 10)
            plsc.store_compressed(wpack.at[pl.ds(nw, _LANES)], packed, mask=m)
            npop = plsc.all_reduce_population_count(m)
            return nw + npop[0]

        nw = lax.fori_loop(0, nchunks, sub_body, 0)

        def batch_body(b, g):
            slot = g % 2

            @pl.when(g >= 2)
            def _():
                pltpu.make_async_copy(
                    out_hbm.at[pl.ds(0, _LANES)],
                    staging.at[pl.ds(0, _LANES)],
                    sem2,
                ).wait()

            msk = (b * _LANES + lane) < nw
            p = wpack[pl.ds(b * _LANES, _LANES)]
            jv = p & 1023
            tv = p >> 10
            jv = jnp.where(msk, jv, jv[0])
            tv = jnp.where(msk, tv, tv[0])
            srot = slot * _LANES + lane
            for e in range(_EMB):
                g_vals = plsc.load_gather(
                    win, [jnp.full((_LANES,), e, jnp.int32), jv]
                )
                plsc.store_scatter(
                    staging,
                    [srot, jnp.full((_LANES,), e, jnp.int32)],
                    g_vals,
                )
            srow[slot, :] = tv
            pltpu.async_copy(
                staging.at[pl.ds(slot * _LANES, _LANES)],
                out_hbm.at[srow.at[slot]],
                sem2,
            )
            return g + 1

        return lax.fori_loop(0, (nw + _LANES - 1) // _LANES, batch_body, gb)

    def step(i, gb):
        gb = process(2 * i, win0, sem0, gb)

        @pl.when(2 * i + 2 < _WPT)
        def _():
            fire(2 * i + 2, win0, sem0)

        gb = process(2 * i + 1, win1, sem1, gb)

        @pl.when(2 * i + 3 < _WPT)
        def _():
            fire(2 * i + 3, win1, sem1)

        return gb

    gb = lax.fori_loop(0, _WPT // 2, step, 0)

    def drain_body(i, carry):
        pltpu.make_async_copy(
            out_hbm.at[pl.ds(0, _LANES)],
            staging.at[pl.ds(0, _LANES)],
            sem2,
        ).wait()
        return carry

    lax.fori_loop(0, jnp.minimum(gb, 2), drain_body, 0)


def _epilogue_body(raw_ref, idx_ref, pe_ref, out_ref):
    rows = raw_ref[:, :_EMB]
    m = (idx_ref[0, 0, :] != 0).astype(jnp.float32).reshape(-1, 1)
    out_ref[...] = rows * m + pe_ref[...]


_EPI_BLK = 256


def _epilogue(raw, idx3, pe):
    return pl.pallas_call(
        _epilogue_body,
        grid=(_NTOK // _EPI_BLK,),
        in_specs=[
            pl.BlockSpec((_EPI_BLK, 128), lambda b: (b, 0)),
            pl.BlockSpec((1, 1, _EPI_BLK), lambda b: (b, 0, 0)),
            pl.BlockSpec((_EPI_BLK, _EMB), lambda b: (b % (_SEQ // _EPI_BLK), 0)),
        ],
        out_specs=pl.BlockSpec((_EPI_BLK, _EMB), lambda b: (b, 0)),
        out_shape=jax.ShapeDtypeStruct((_NTOK, _EMB), jnp.float32),
    )(raw, idx3, pe)


def kernel(input, table):
    idx_flat = input.reshape(_NTOK)
    raw = _gather_sc(table.T, idx_flat)
    idx3 = idx_flat.reshape(_NTOK // _EPI_BLK, 1, _EPI_BLK)
    pe = jnp.asarray(_PE)
    out = _epilogue(raw, idx3, pe)
    return out.reshape(_BATCH, _SEQ, _EMB)


# speedup vs baseline: 1.1228x; 1.0244x over previous
import functools

import numpy as np
import jax
import jax.numpy as jnp
from jax import lax
from jax.experimental import pallas as pl
from jax.experimental.pallas import tpu as pltpu
from jax.experimental.pallas import tpu_sc as plsc

_VOCAB = 1000000
_EMB = 64
_SEQ = 2048
_BATCH = 4
_NTOK = _BATCH * _SEQ

_NC = 2
_NS = 16
_NW = _NC * _NS
_LANES = 16

_WCOLS = 512
_NWIN = -(-_VOCAB // _WCOLS)
_WPT = (-(-_NWIN // _NW) + 1) // 2 * 2
_LAST_COL0 = -(-(_VOCAB - _WCOLS) // 128) * 128
_OUT_ROWS = _NTOK


def _pe_host(seq: int, d: int) -> np.ndarray:
    pos = np.arange(seq, dtype=np.float64)[:, None]
    index = np.arange(d, dtype=np.float64)[None, :]
    tmp = pos / np.power(10000.0, index / float(d))
    pe = np.zeros((seq, d), dtype=np.float64)
    pe[:, 0::2] = np.sin(tmp[:, 0::2])
    pe[:, 1::2] = np.cos(tmp[:, 1::2])
    return pe.astype(np.float32)


_PE = _pe_host(_SEQ, _EMB)

_mesh = plsc.VectorSubcoreMesh(core_axis_name="c", subcore_axis_name="s")


@functools.partial(
    pl.kernel,
    mesh=_mesh,
    compiler_params=pltpu.CompilerParams(needs_layout_passes=False),
    out_type=jax.ShapeDtypeStruct((_OUT_ROWS, 128), jnp.float32),
    scratch_types=[
        pltpu.VMEM((_NTOK,), jnp.int32),
        pltpu.VMEM((_NTOK,), jnp.int32),
        pltpu.VMEM((_NTOK,), jnp.int32),
        pltpu.VMEM((_NTOK,), jnp.int32),
        pltpu.VMEM((_EMB, _WCOLS), jnp.float32),
        pltpu.VMEM((_EMB, _WCOLS), jnp.float32),
        pltpu.VMEM((2 * _LANES, 128), jnp.float32),
        pltpu.VMEM((2, _LANES), jnp.int32),
        pltpu.SemaphoreType.DMA,
        pltpu.SemaphoreType.DMA,
        pltpu.SemaphoreType.DMA,
    ],
)
def _gather_sc(tablet_hbm, idx_hbm, out_hbm, idx_v, vlist, tlist, wpack,
               win0, win1, staging, srow, sem0, sem1, sem2):
    wid = lax.axis_index("s") * _NC + lax.axis_index("c")

    def fire(k, win, sem):
        w_glob = wid + _NW * k
        col0 = jnp.minimum(w_glob * _WCOLS, _LAST_COL0)
        col0 = pl.multiple_of(col0, 128)
        pltpu.async_copy(tablet_hbm.at[:, pl.ds(col0, _WCOLS)], win, sem)

    fire(0, win0, sem0)
    fire(1, win1, sem1)

    pltpu.sync_copy(idx_hbm, idx_v)

    lane = lax.broadcasted_iota(jnp.int32, (_LANES,), 0)

    def bucket_body(c, cnt):
        v = idx_v[pl.ds(c * _LANES, _LANES)]
        m = ((v >> 9) & (_NW - 1)) == wid
        plsc.store_compressed(vlist.at[pl.ds(cnt, _LANES)], v, mask=m)
        plsc.store_compressed(
            tlist.at[pl.ds(cnt, _LANES)], c * _LANES + lane, mask=m
        )
        npop = plsc.all_reduce_population_count(m)
        return cnt + npop[0]

    cnt = lax.fori_loop(0, _NTOK // _LANES, bucket_body, 0)
    nchunks = (cnt + _LANES - 1) // _LANES

    def process(k, win, sem, gb):
        pltpu.make_async_copy(
            tablet_hbm.at[:, pl.ds(0, _WCOLS)], win, sem
        ).wait()
        w_glob = wid + _NW * k
        dma_col0 = jnp.minimum(w_glob * _WCOLS, _LAST_COL0)

        def sub_body(c, nw):
            valid = (c * _LANES + lane) < cnt
            v = vlist[pl.ds(c * _LANES, _LANES)]
            t = tlist[pl.ds(c * _LANES, _LANES)]
            m = valid & ((v >> 9) == w_glob)
            packed = (v - dma_col0) | (t << 10)
            plsc.store_compressed(wpack.at[pl.ds(nw, _LANES)], packed, mask=m)
            npop = plsc.all_reduce_population_count(m)
            return nw + npop[0]

        nw = lax.fori_loop(0, nchunks, sub_body, 0)

        def batch_body(b, g):
            slot = g % 2

            @pl.when(g >= 2)
            def _():
                pltpu.make_async_copy(
                    out_hbm.at[pl.ds(0, _LANES)],
                    staging.at[pl.ds(0, _LANES)],
                    sem2,
                ).wait()

            msk = (b * _LANES + lane) < nw
            p = wpack[pl.ds(b * _LANES, _LANES)]
            jv = p & 1023
            tv = p >> 10
            jv = jnp.where(msk, jv, jv[0])
            tv = jnp.where(msk, tv, tv[0])
            srot = slot * _LANES + lane
            for e in range(_EMB):
                g_vals = plsc.load_gather(
                    win, [jnp.full((_LANES,), e, jnp.int32), jv]
                )
                plsc.store_scatter(
                    staging,
                    [srot, jnp.full((_LANES,), e, jnp.int32)],
                    g_vals,
                )
            srow[slot, :] = tv
            pltpu.async_copy(
                staging.at[pl.ds(slot * _LANES, _LANES)],
                out_hbm.at[srow.at[slot]],
                sem2,
            )
            return g + 1

        return lax.fori_loop(0, (nw + _LANES - 1) // _LANES, batch_body, gb)

    def step(i, gb):
        gb = process(2 * i, win0, sem0, gb)

        @pl.when(2 * i + 2 < _WPT)
        def _():
            fire(2 * i + 2, win0, sem0)

        gb = process(2 * i + 1, win1, sem1, gb)

        @pl.when(2 * i + 3 < _WPT)
        def _():
            fire(2 * i + 3, win1, sem1)

        return gb

    gb = lax.fori_loop(0, _WPT // 2, step, 0)

    def drain_body(i, carry):
        pltpu.make_async_copy(
            out_hbm.at[pl.ds(0, _LANES)],
            staging.at[pl.ds(0, _LANES)],
            sem2,
        ).wait()
        return carry

    lax.fori_loop(0, jnp.minimum(gb, 2), drain_body, 0)


def _epilogue_body(raw_ref, idx_ref, pe_ref, out_ref):
    rows = raw_ref[:, :_EMB]
    m = (idx_ref[0, 0, :] != 0).astype(jnp.float32).reshape(-1, 1)
    out_ref[0, :, :] = jnp.transpose(rows * m + pe_ref[...], (1, 0))


_EPI_BLK = 256


def _epilogue(raw, idx3, pe):
    return pl.pallas_call(
        _epilogue_body,
        grid=(_NTOK // _EPI_BLK,),
        in_specs=[
            pl.BlockSpec((_EPI_BLK, 128), lambda b: (b, 0)),
            pl.BlockSpec((1, 1, _EPI_BLK), lambda b: (b, 0, 0)),
            pl.BlockSpec((_EPI_BLK, _EMB), lambda b: (b % (_SEQ // _EPI_BLK), 0)),
        ],
        out_specs=pl.BlockSpec(
            (1, _EMB, _EPI_BLK),
            lambda b: (b // (_SEQ // _EPI_BLK), 0, b % (_SEQ // _EPI_BLK)),
        ),
        out_shape=jax.ShapeDtypeStruct((_BATCH, _EMB, _SEQ), jnp.float32),
    )(raw, idx3, pe)


def kernel(input, table):
    idx_flat = input.reshape(_NTOK)
    raw = _gather_sc(table.T, idx_flat)
    idx3 = idx_flat.reshape(_NTOK // _EPI_BLK, 1, _EPI_BLK)
    pe = jnp.asarray(_PE)
    out_t = _epilogue(raw, idx3, pe)
    return jnp.transpose(out_t, (0, 2, 1))
